# baseline (speedup 1.0000x reference)
import functools

import jax
import jax.numpy as jnp
from jax import lax
from jax.experimental import pallas as pl
from jax.experimental.pallas import tpu as pltpu
from jax.experimental.pallas import tpu_sc as plsc

VOCAB = 100000
EMB = 128
B = 4096
L = 50
TOT = B * L
NC = 2
NS = 16
NW = NC * NS
PER_W = TOT // NW
C = 128
NCH = PER_W // C
NB = 2
NG = NCH // NB

_mesh = plsc.VectorSubcoreMesh(core_axis_name="c", subcore_axis_name="s")


@functools.partial(
    pl.kernel,
    out_type=jax.ShapeDtypeStruct((TOT, EMB), jnp.float32),
    mesh=_mesh,
    scratch_types=[
        pltpu.VMEM((NCH, C), jnp.int32),
        [pltpu.VMEM((C, EMB), jnp.float32) for _ in range(NB)],
        [pltpu.SemaphoreType.DMA for _ in range(NB)],
        [pltpu.SemaphoreType.DMA for _ in range(NB)],
    ],
)
def _gather_kernel(table_hbm, idx_hbm, out_hbm, idx_v, bufs, gsems, osems):
    wid = lax.axis_index("s") * NC + lax.axis_index("c")
    wbase = wid * PER_W
    pltpu.sync_copy(idx_hbm.at[wid], idx_v)

    def group(gi, carry):
        for b in range(NB):
            g = gi * NB + b

            @pl.when(gi > 0)
            def _():
                pltpu.make_async_copy(
                    bufs[b], out_hbm.at[pl.ds(wbase, C)], osems[b]
                ).wait()

            pltpu.async_copy(table_hbm.at[idx_v.at[g]], bufs[b], gsems[b])
        for b in range(NB):
            g = gi * NB + b
            pltpu.make_async_copy(
                table_hbm.at[idx_v.at[g]], bufs[b], gsems[b]
            ).wait()
            pltpu.async_copy(bufs[b], out_hbm.at[pl.ds(wbase + g * C, C)],
                             osems[b])
        return carry

    lax.fori_loop(0, NG, group, 0)
    for b in range(NB):
        pltpu.make_async_copy(
            bufs[b], out_hbm.at[pl.ds(wbase, C)], osems[b]
        ).wait()


def kernel(input_ids, table):
    idx = input_ids.astype(jnp.int32).T.reshape(NW, NCH, C)
    out = _gather_kernel(table, idx)
    return out.reshape(L, B, EMB).transpose(1, 0, 2)
